# Initial kernel scaffold; baseline (speedup 1.0000x reference)
import jax
import jax.numpy as jnp
from jax.experimental import pallas as pl


def kernel(y_true, y_pred, B):
    raise NotImplementedError("write your pallas kernel here")


# speedup vs baseline: 5.1853x; 5.1853x over previous
import functools

import jax
import jax.numpy as jnp
from jax import lax
from jax.experimental import pallas as pl
from jax.experimental.pallas import tpu as pltpu
from jax.experimental.pallas import tpu_sc as plsc

C = 24
L = 16
NC = 2
NS = 16
NW = NC * NS
T = 1024


def _sc_score(yt2d, yp2d, Bflat):
    n_steps = yt2d.shape[0]
    mesh = plsc.VectorSubcoreMesh(core_axis_name="core",
                                  subcore_axis_name="subcore")
    cparams = pltpu.CompilerParams(needs_layout_passes=False)

    @functools.partial(
        pl.kernel,
        out_type=jax.ShapeDtypeStruct((NW, L), jnp.float32),
        mesh=mesh,
        scratch_types=[
            pltpu.VMEM((C * C,), jnp.float32),
            pltpu.VMEM((L,), jnp.float32),
        ],
        compiler_params=cparams,
    )
    def kern(yt_hbm, yp_hbm, b_hbm, out_hbm, bv, accv):
        wid = lax.axis_index("subcore") * NC + lax.axis_index("core")
        pltpu.sync_copy(b_hbm, bv)
        accv[...] = jnp.zeros((L,), jnp.float32)
        iota = lax.iota(jnp.int32, L)
        zero16 = jnp.zeros((L,), jnp.int32)

        def body(yt_vmem, yp_vmem):
            def group(g, acc):
                tv = yt_vmem[0, pl.ds(g * L, L)]
                bbase = tv * C
                pbase = (g * L + iota) * C
                for c in range(C):
                    w = plsc.load_gather(bv, [bbase + c])
                    p = plsc.load_gather(yp_vmem, [zero16, pbase + c])
                    acc = acc + w * p
                return acc

            accv[...] = lax.fori_loop(0, T // L, group, accv[...])

        pltpu.emit_pipeline(
            body,
            grid=(n_steps,),
            in_specs=[
                pl.BlockSpec((1, T), lambda i: (i, 0)),
                pl.BlockSpec((1, T * C), lambda i: (i, 0)),
            ],
            out_specs=[],
            core_axis_name=("core", "subcore"),
            dimension_semantics=(pltpu.PARALLEL,),
        )(yt_hbm, yp_hbm)

        pltpu.sync_copy(accv, out_hbm.at[wid])

    return kern(yt2d, yp2d, Bflat)


def kernel(y_true, y_pred, B):
    n = y_true.shape[0] * y_true.shape[1]
    yt2d = y_true.reshape(n // T, T)
    yp2d = y_pred.reshape(n // T, T * C)
    partials = _sc_score(yt2d, yp2d, B.reshape(-1))
    return jnp.sum(partials)


# speedup vs baseline: 5.2284x; 1.0083x over previous
import functools

import jax
import jax.numpy as jnp
from jax import lax
from jax.experimental import pallas as pl
from jax.experimental.pallas import tpu as pltpu
from jax.experimental.pallas import tpu_sc as plsc

C = 24
L = 16
NC = 2
NS = 16
NW = NC * NS
T = 1024


def _sc_score(yt2d, yp2d, Bflat):
    n_steps = yt2d.shape[0]
    mesh = plsc.VectorSubcoreMesh(core_axis_name="core",
                                  subcore_axis_name="subcore")
    cparams = pltpu.CompilerParams(needs_layout_passes=False)

    @functools.partial(
        pl.kernel,
        out_type=jax.ShapeDtypeStruct((NW, L), jnp.float32),
        mesh=mesh,
        scratch_types=[
            pltpu.VMEM((C * C,), jnp.float32),
            pltpu.VMEM((C * C,), jnp.float32),
            pltpu.VMEM((L,), jnp.float32),
        ],
        compiler_params=cparams,
    )
    def kern(yt_hbm, yp_hbm, b_hbm, out_hbm, bv, sv, accv):
        wid = lax.axis_index("subcore") * NC + lax.axis_index("core")
        pltpu.sync_copy(b_hbm, bv)
        zeros16 = jnp.zeros((L,), jnp.float32)
        for k in range(C * C // L):
            sv[pl.ds(k * L, L)] = zeros16

        iota = lax.iota(jnp.int32, L)
        hsel = [jnp.zeros((L,), jnp.int32),
                jnp.where(iota < 8, 0, 1).astype(jnp.int32),
                jnp.ones((L,), jnp.int32)]
        csel = [iota,
                jnp.where(iota < 8, iota + 16, iota - 8).astype(jnp.int32),
                iota + 8]

        def body(yt_vmem, yp_vmem):
            def group(g, carry):
                tok24 = yt_vmem[0, pl.ds(g * L, L)] * C
                base = g * (L * C)
                for j in range(8):
                    for v in range(3):
                        k = 3 * j + v
                        m = jnp.full((L,), 2 * j, jnp.int32) + hsel[v]
                        t24 = lax.gather(
                            tok24, m[:, None],
                            lax.GatherDimensionNumbers(
                                offset_dims=(), collapsed_slice_dims=(0,),
                                start_index_map=(0,)),
                            (1,),
                            mode=lax.GatherScatterMode.PROMISE_IN_BOUNDS)
                        p = yp_vmem[0, pl.ds(base + k * L, L)]
                        plsc.addupdate_scatter(sv, [t24 + csel[v]], p)
                return carry

            lax.fori_loop(0, T // L, group, 0)

        pltpu.emit_pipeline(
            body,
            grid=(n_steps,),
            in_specs=[
                pl.BlockSpec((1, T), lambda i: (i, 0)),
                pl.BlockSpec((1, T * C), lambda i: (i, 0)),
            ],
            out_specs=[],
            core_axis_name=("core", "subcore"),
            dimension_semantics=(pltpu.PARALLEL,),
        )(yt_hbm, yp_hbm)

        acc = jnp.zeros((L,), jnp.float32)
        for k in range(C * C // L):
            acc = acc + sv[pl.ds(k * L, L)] * bv[pl.ds(k * L, L)]
        accv[...] = acc
        pltpu.sync_copy(accv, out_hbm.at[wid])

    return kern(yt2d, yp2d, Bflat)


def kernel(y_true, y_pred, B):
    n = y_true.shape[0] * y_true.shape[1]
    yt2d = y_true.reshape(n // T, T)
    yp2d = y_pred.reshape(n // T, T * C)
    partials = _sc_score(yt2d, yp2d, B.reshape(-1))
    return jnp.sum(partials)


# speedup vs baseline: 7.3574x; 1.4072x over previous
import functools

import jax
import jax.numpy as jnp
from jax import lax
from jax.experimental import pallas as pl
from jax.experimental.pallas import tpu as pltpu
from jax.experimental.pallas import tpu_sc as plsc

C = 24
L = 16
NC = 2
NS = 16
NW = NC * NS
SEQ = 200


def _bcast(vec, lane):
    m = jnp.full((L,), lane, jnp.int32)
    return lax.gather(
        vec, m[:, None],
        lax.GatherDimensionNumbers(offset_dims=(), collapsed_slice_dims=(0,),
                                   start_index_map=(0,)),
        (1,), mode=lax.GatherScatterMode.PROMISE_IN_BOUNDS)


def _sc_score(ytp, yp3, bpad):
    n_steps = ytp.shape[0]
    mesh = plsc.VectorSubcoreMesh(core_axis_name="core",
                                  subcore_axis_name="subcore")
    cparams = pltpu.CompilerParams(needs_layout_passes=False,
                                   use_tc_tiling_on_sc=True)

    @functools.partial(
        pl.kernel,
        out_type=jax.ShapeDtypeStruct((NW, L), jnp.float32),
        mesh=mesh,
        scratch_types=[
            pltpu.VMEM((8, 128), jnp.float32),
            pltpu.VMEM((C * C,), jnp.float32),
            pltpu.VMEM((L,), jnp.float32),
        ],
        compiler_params=cparams,
    )
    def kern(yt_hbm, yp_hbm, b_hbm, out_hbm, bv, sv, accv):
        wid = lax.axis_index("subcore") * NC + lax.axis_index("core")
        pltpu.sync_copy(b_hbm, bv)
        zeros16 = jnp.zeros((L,), jnp.float32)
        for k in range(C * C // L):
            sv[pl.ds(k * L, L)] = zeros16

        iota = lax.iota(jnp.int32, L)
        himask = iota >= 8

        def body(yt_vmem, yp_vmem):
            for g in range(13):
                r, l0 = divmod(g * L, 128)
                tok24 = yt_vmem[0, r, pl.ds(l0, L)]
                for j in range(L if g < 12 else 8):
                    s = g * L + j
                    t24 = _bcast(tok24, j)
                    v0 = yp_vmem[0, s, pl.ds(0, L)]
                    plsc.addupdate_scatter(sv, [t24 + iota], v0)
                    v1 = yp_vmem[0, s, pl.ds(8, L)]
                    plsc.addupdate_scatter(sv, [t24 + (8 + iota)], v1,
                                           mask=himask)

        pltpu.emit_pipeline(
            body,
            grid=(n_steps,),
            in_specs=[
                pl.BlockSpec((1, 2, 128), lambda i: (i, 0, 0)),
                pl.BlockSpec((1, SEQ, C), lambda i: (i, 0, 0)),
            ],
            out_specs=[],
            core_axis_name=("core", "subcore"),
            dimension_semantics=(pltpu.PARALLEL,),
        )(yt_hbm, yp_hbm)

        acc = jnp.zeros((L,), jnp.float32)
        for k in range(C * C // L):
            f = k * L
            acc = acc + sv[pl.ds(f, L)] * bv[f // 128, pl.ds(f % 128, L)]
        accv[...] = acc
        pltpu.sync_copy(accv, out_hbm.at[wid])

    return kern(ytp, yp3, bpad)


def kernel(y_true, y_pred, B):
    yt24 = y_true.astype(jnp.int32) * C
    ytp = jnp.pad(yt24, ((0, 0), (0, 256 - SEQ))).reshape(-1, 2, 128)
    bpad = jnp.pad(B.reshape(-1), (0, 8 * 128 - C * C)).reshape(8, 128)
    partials = _sc_score(ytp, y_pred, bpad)
    return jnp.sum(partials)


# speedup vs baseline: 53.2864x; 7.2426x over previous
import functools

import jax
import jax.numpy as jnp
from jax import lax
from jax.experimental import pallas as pl
from jax.experimental.pallas import tpu as pltpu
from jax.experimental.pallas import tpu_sc as plsc

C = 24
BROW = 25
L = 16
NC = 2
NS = 16
NW = NC * NS
BCH = 256
SOCT = 8
NACC = 4


def _sc_score(ytp, ypt, bpad):
    n_seq, _, n_batch = ytp.shape
    grid = (n_batch // BCH) * n_seq
    nb = n_batch // BCH
    mesh = plsc.VectorSubcoreMesh(core_axis_name="core",
                                  subcore_axis_name="subcore")
    cparams = pltpu.CompilerParams(needs_layout_passes=False,
                                   use_tc_tiling_on_sc=True)

    @functools.partial(
        pl.kernel,
        out_type=jax.ShapeDtypeStruct((NW, L), jnp.float32),
        mesh=mesh,
        scratch_types=[
            pltpu.VMEM((BROW * C + 40,), jnp.float32),
            pltpu.VMEM((NACC, L), jnp.float32),
        ],
        compiler_params=cparams,
    )
    def kern(yt_hbm, yp_hbm, b_hbm, out_hbm, bv, accv):
        wid = lax.axis_index("subcore") * NC + lax.axis_index("core")
        pltpu.sync_copy(b_hbm, bv)
        zeros16 = jnp.zeros((L,), jnp.float32)
        for a in range(NACC):
            accv[a] = zeros16

        def body(yt_vmem, yp_vmem):
            @pl.loop(0, BCH // L)
            def _(g):
                l0 = g * L
                for r in range(SOCT):
                    t25 = yt_vmem[0, r, pl.ds(l0, L)] * BROW
                    acc = [accv[a] for a in range(NACC)]
                    for c in range(C):
                        w = plsc.load_gather(bv, [t25 + c])
                        p = yp_vmem[r, c, pl.ds(l0, L)]
                        acc[c % NACC] = acc[c % NACC] + w * p
                    for a in range(NACC):
                        accv[a] = acc[a]

        pltpu.emit_pipeline(
            body,
            grid=(grid,),
            in_specs=[
                pl.BlockSpec((1, SOCT, BCH),
                             lambda i: (i % n_seq, 0, i // n_seq)),
                pl.BlockSpec((SOCT, C, BCH),
                             lambda i: (i % n_seq, 0, i // n_seq)),
            ],
            out_specs=[],
            core_axis_name=("core", "subcore"),
            dimension_semantics=(pltpu.PARALLEL,),
        )(yt_hbm, yp_hbm)

        acc = (accv[0] + accv[1]) + (accv[2] + accv[3])
        accv[0] = acc
        pltpu.sync_copy(accv.at[0], out_hbm.at[wid])

    return kern(ytp, ypt, bpad)


def kernel(y_true, y_pred, B):
    seq = y_true.shape[1]
    ypt = jnp.transpose(y_pred, (1, 2, 0))
    ytp = jnp.transpose(y_true.astype(jnp.int32), (1, 0)).reshape(
        seq // SOCT, SOCT, -1)
    bpad = jnp.pad(B, ((0, 0), (0, BROW - C))).reshape(-1)
    bpad = jnp.pad(bpad, (0, 640 - BROW * C))
    partials = _sc_score(ytp, ypt, bpad)
    return jnp.sum(partials)
